# baseline (speedup 1.0000x reference)
import math

import numpy as np
import jax
import jax.numpy as jnp
from jax.experimental import pallas as pl
from jax.experimental.pallas import tpu as pltpu

M = 65536
C = 1024
BM = 512
ALPHA = 0.7
BETA = 0.3
EPS = 1e-06
K = 13108


def _xla_rowsum(a):
    t = a[:, 0:128]
    for c in range(1, 8):
        t = t + a[:, 128 * c:128 * (c + 1)]
    u = t[:, 0:8]
    for k in range(1, 16):
        u = u + t[:, 8 * k:8 * (k + 1)]
    u = u[:, :4] + u[:, 4:]
    u = u[:, :2] + u[:, 2:]
    return u[:, 0] + u[:, 1]


def _entropy_kernel(x_ref, out_ref):
    x = x_ref[...]
    x = jnp.nan_to_num(x, nan=0.0, posinf=0.0, neginf=0.0)
    m = jnp.max(x, axis=1, keepdims=True)
    e = jnp.exp(x - m)
    z = _xla_rowsum(e)
    p = e / z[:, None]
    lp = jnp.log(p + EPS)
    ent = -_xla_rowsum(p * lp)
    out_ref[0, 0, :] = ent


ENT_SCALE = float(np.float32(np.float32(ALPHA) / np.float32(math.log(C) + EPS)))


def _combine_kernel(ent_ref, geo_ref, out_ref):
    g = geo_ref[...]
    g = jnp.nan_to_num(g, nan=0.0, posinf=0.0, neginf=0.0)
    gmin = jnp.min(g)
    gmax = jnp.max(g)
    gn = (g - gmin) / (gmax - gmin + EPS)
    out_ref[...] = ent_ref[...] * ENT_SCALE + BETA * gn


def _scores(coarse_logits, handcrafted_features):
    nb = M // BM
    ent = pl.pallas_call(
        _entropy_kernel,
        grid=(nb,),
        in_specs=[pl.BlockSpec((BM, C), lambda i: (i, 0))],
        out_specs=pl.BlockSpec((1, 1, BM), lambda i: (i, 0, 0)),
        out_shape=jax.ShapeDtypeStruct((nb, 1, BM), jnp.float32),
    )(coarse_logits)
    ent2d = ent.reshape(M // 128, 128)
    geo2d = handcrafted_features[:, 2].reshape(M // 128, 128)
    total = pl.pallas_call(
        _combine_kernel,
        in_specs=[
            pl.BlockSpec((M // 128, 128), lambda: (0, 0)),
            pl.BlockSpec((M // 128, 128), lambda: (0, 0)),
        ],
        out_specs=pl.BlockSpec((M // 128, 128), lambda: (0, 0)),
        out_shape=jax.ShapeDtypeStruct((M // 128, 128), jnp.float32),
    )(ent2d, geo2d)
    return total.reshape(M)


def kernel(coarse_logits, handcrafted_features):
    total_scores = _scores(coarse_logits, handcrafted_features)
    _, hard_sp_indices = jax.lax.top_k(total_scores, K)
    return (hard_sp_indices, total_scores)
